# baseline (speedup 1.0000x reference)
import functools

import jax
import jax.numpy as jnp
from jax import lax
from jax.experimental import pallas as pl
from jax.experimental.pallas import tpu as pltpu
from jax.experimental.pallas import tpu_sc as plsc

_VOCAB = 119
_D = 512
_BATCH, _SEQ = 4096, 50
_SEQP = 64

_NC, _NS = 2, 16
_NW = _NC * _NS
_B_PER_W = _BATCH // _NW


def _proj_body(t1, w1, b1, t2, w2, b2, t3, w3, b3, o1, o2, o3):
    o1[...] = jnp.dot(t1[...], w1[...], preferred_element_type=jnp.float32) + b1[...]
    o2[...] = jnp.dot(t2[...], w2[...], preferred_element_type=jnp.float32) + b2[...]
    o3[...] = jnp.dot(t3[...], w3[...], preferred_element_type=jnp.float32) + b3[...]


def _project_tables(t1, w1, b1, t2, w2, b2, t3, w3, b3):
    out = [jax.ShapeDtypeStruct((_VOCAB, _D), jnp.float32)] * 3
    return pl.pallas_call(_proj_body, out_shape=out)(
        t1, w1, b1.reshape(1, _D), t2, w2, b2.reshape(1, _D),
        t3, w3, b3.reshape(1, _D))


_mesh = plsc.VectorSubcoreMesh(core_axis_name="c", subcore_axis_name="s")


@functools.partial(
    pl.kernel,
    mesh=_mesh,
    out_type=[jax.ShapeDtypeStruct((_BATCH, _SEQ, _D), jnp.float32)] * 3,
    scratch_types=[
        pltpu.VMEM((_B_PER_W, _SEQP), jnp.int32),
        pltpu.VMEM((_SEQ, _D), jnp.float32),
        pltpu.VMEM((_SEQ, _D), jnp.float32),
        pltpu.VMEM((_SEQ, _D), jnp.float32),
        pltpu.SemaphoreType.DMA,
        pltpu.SemaphoreType.DMA,
    ],
    compiler_params=pltpu.CompilerParams(use_tc_tiling_on_sc=True),
)
def _gather_all(p1, p2, p3, idx_hbm, o1, o2, o3,
                idx_v, r0, r1, r2, gsem, wsem):
    wid = lax.axis_index("s") * _NC + lax.axis_index("c")
    tbls = (p1, p2, p3)
    outs = (o1, o2, o3)
    bufs = (r0, r1, r2)

    b0 = wid * _B_PER_W
    pltpu.sync_copy(idx_hbm.at[pl.ds(b0, _B_PER_W)], idx_v)

    def batch_body(b, carry):
        idx_c = idx_v.at[b, pl.ds(0, _SEQ)]
        for t in range(3):
            @pl.when(b > 0)
            def _drain():
                pltpu.make_async_copy(bufs[t], outs[t].at[0], wsem).wait()

            pltpu.async_copy(tbls[t].at[idx_c], bufs[t], gsem).wait()
            pltpu.async_copy(bufs[t], outs[t].at[b0 + b], wsem)
        return carry

    lax.fori_loop(0, _B_PER_W, batch_body, 0)
    for t in range(3):
        pltpu.make_async_copy(bufs[t], outs[t].at[0], wsem).wait()


def kernel(Z, table_mat2vec, table_magpie, table_oliynyk,
           W_mat2vec, b_mat2vec, W_magpie, b_magpie, W_oliynyk, b_oliynyk):
    p1, p2, p3 = _project_tables(
        table_mat2vec, W_mat2vec, b_mat2vec,
        table_magpie, W_magpie, b_magpie,
        table_oliynyk, W_oliynyk, b_oliynyk)
    zp = jnp.pad(Z, ((0, 0), (0, _SEQP - _SEQ)))
    return _gather_all(p1, p2, p3, zp)


# speedup vs baseline: 1.0005x; 1.0005x over previous
import functools

import jax
import jax.numpy as jnp
from jax import lax
from jax.experimental import pallas as pl
from jax.experimental.pallas import tpu as pltpu
from jax.experimental.pallas import tpu_sc as plsc

_VOCAB = 119
_D = 512
_BATCH, _SEQ = 4096, 50
_SEQP = 64

_NC, _NS = 2, 16
_NW = _NC * _NS
_B_PER_W = _BATCH // _NW


def _proj_body(t1, w1, b1, t2, w2, b2, t3, w3, b3, o1, o2, o3):
    o1[...] = jnp.dot(t1[...], w1[...], preferred_element_type=jnp.float32) + b1[...]
    o2[...] = jnp.dot(t2[...], w2[...], preferred_element_type=jnp.float32) + b2[...]
    o3[...] = jnp.dot(t3[...], w3[...], preferred_element_type=jnp.float32) + b3[...]


def _project_tables(t1, w1, b1, t2, w2, b2, t3, w3, b3):
    out = [jax.ShapeDtypeStruct((_VOCAB, _D), jnp.float32)] * 3
    return pl.pallas_call(_proj_body, out_shape=out)(
        t1, w1, b1.reshape(1, _D), t2, w2, b2.reshape(1, _D),
        t3, w3, b3.reshape(1, _D))


_mesh = plsc.VectorSubcoreMesh(core_axis_name="c", subcore_axis_name="s")


@functools.partial(
    pl.kernel,
    mesh=_mesh,
    out_type=[jax.ShapeDtypeStruct((_BATCH, _SEQ, _D), jnp.float32)] * 3,
    scratch_types=[
        pltpu.VMEM((_B_PER_W, _SEQP), jnp.int32),
        pltpu.VMEM((_SEQ, _D), jnp.float32),
        pltpu.VMEM((_SEQ, _D), jnp.float32),
        pltpu.VMEM((_SEQ, _D), jnp.float32),
        pltpu.SemaphoreType.DMA,
        pltpu.SemaphoreType.DMA,
    ],
    compiler_params=pltpu.CompilerParams(use_tc_tiling_on_sc=True),
)
def _gather_all(p1, p2, p3, idx_hbm, o1, o2, o3,
                idx_v, r0, r1, r2, gsem, wsem):
    wid = lax.axis_index("s") * _NC + lax.axis_index("c")
    tbls = (p1, p2, p3)
    outs = (o1, o2, o3)
    bufs = (r0, r1, r2)

    b0 = wid * _B_PER_W
    pltpu.sync_copy(idx_hbm.at[pl.ds(b0, _B_PER_W)], idx_v)

    def batch_body(b, carry):
        idx_c = idx_v.at[b, pl.ds(0, _SEQ)]
        for t in range(3):
            @pl.when(b > 0)
            def _drain():
                pltpu.make_async_copy(bufs[t], outs[t].at[0], wsem).wait()

            pltpu.async_copy(tbls[t].at[idx_c], bufs[t], gsem).wait()
            pltpu.async_copy(bufs[t], outs[t].at[b0 + b], wsem)
        return carry

    lax.fori_loop(0, _B_PER_W, batch_body, 0)
    for t in range(3):
        pltpu.make_async_copy(bufs[t], outs[t].at[0], wsem).wait()


def kernel(Z, table_mat2vec, table_magpie, table_oliynyk,
           W_mat2vec, b_mat2vec, W_magpie, b_magpie, W_oliynyk, b_oliynyk):
    p1, p2, p3 = _project_tables(
        table_mat2vec, W_mat2vec, b_mat2vec,
        table_magpie, W_magpie, b_magpie,
        table_oliynyk, W_oliynyk, b_oliynyk)
    zp = jnp.pad(Z, ((0, 0), (0, _SEQP - _SEQ)))
    return tuple(_gather_all(p1, p2, p3, zp))


# speedup vs baseline: 1.0233x; 1.0228x over previous
import functools

import jax
import jax.numpy as jnp
from jax import lax
from jax.experimental import pallas as pl
from jax.experimental.pallas import tpu as pltpu
from jax.experimental.pallas import tpu_sc as plsc

_VOCAB = 119
_VPAD = 128
_D = 512
_BATCH, _SEQ = 4096, 50
_SEQ_SC = 48
_SEQ_TC = _SEQ - _SEQ_SC
_SEQ_TCP = 8

_NC, _NS = 2, 16
_NW = _NC * _NS
_B_PER_W = _BATCH // _NW

_FIX_GRP = 64


def _proj_body(t1, w1, b1, t2, w2, b2, t3, w3, b3, o1, o2, o3):
    o1[...] = jnp.dot(t1[...], w1[...], preferred_element_type=jnp.float32) + b1[...]
    o2[...] = jnp.dot(t2[...], w2[...], preferred_element_type=jnp.float32) + b2[...]
    o3[...] = jnp.dot(t3[...], w3[...], preferred_element_type=jnp.float32) + b3[...]


def _project_tables(t1, w1, b1, t2, w2, b2, t3, w3, b3):
    out = [jax.ShapeDtypeStruct((_VOCAB, _D), jnp.float32)] * 3
    return pl.pallas_call(_proj_body, out_shape=out)(
        t1, w1, b1.reshape(1, _D), t2, w2, b2.reshape(1, _D),
        t3, w3, b3.reshape(1, _D))


_mesh = plsc.VectorSubcoreMesh(core_axis_name="c", subcore_axis_name="s")


@functools.partial(
    pl.kernel,
    mesh=_mesh,
    out_type=[jax.ShapeDtypeStruct((_BATCH, _SEQ, _D), jnp.float32)] * 3,
    scratch_types=[
        pltpu.VMEM((_B_PER_W, _SEQ_SC), jnp.int32),
        pltpu.VMEM((_SEQ_SC, _D), jnp.float32),
        pltpu.VMEM((_SEQ_SC, _D), jnp.float32),
        pltpu.VMEM((_SEQ_SC, _D), jnp.float32),
        pltpu.SemaphoreType.DMA,
        pltpu.SemaphoreType.DMA,
    ],
    compiler_params=pltpu.CompilerParams(use_tc_tiling_on_sc=True),
)
def _gather_sc(p1, p2, p3, idx_hbm, o1, o2, o3,
               idx_v, r0, r1, r2, gsem, wsem):
    wid = lax.axis_index("s") * _NC + lax.axis_index("c")
    tbls = (p1, p2, p3)
    outs = (o1, o2, o3)
    bufs = (r0, r1, r2)

    b0 = wid * _B_PER_W
    pltpu.sync_copy(idx_hbm.at[pl.ds(b0, _B_PER_W)], idx_v)

    def batch_body(b, carry):
        idx_c = idx_v.at[b]
        for t in range(3):
            @pl.when(b > 0)
            def _drain():
                pltpu.make_async_copy(
                    bufs[t], outs[t].at[0, pl.ds(0, _SEQ_SC)], wsem).wait()

            pltpu.async_copy(tbls[t].at[idx_c], bufs[t], gsem).wait()
            pltpu.async_copy(
                bufs[t], outs[t].at[b0 + b, pl.ds(0, _SEQ_SC)], wsem)
        return carry

    lax.fori_loop(0, _B_PER_W, batch_body, 0)
    for t in range(3):
        pltpu.make_async_copy(
            bufs[t], outs[t].at[0, pl.ds(0, _SEQ_SC)], wsem).wait()


def _fix_body(zb, _a1, _a2, _a3, p1, p2, p3, o1, o2, o3):
    zcol = zb[0].reshape(_FIX_GRP * _SEQ_TCP, 1)
    iota_v = lax.broadcasted_iota(jnp.int32, (_FIX_GRP * _SEQ_TCP, _VPAD), 1)
    oh = (zcol == iota_v).astype(jnp.float32)
    for p, o in ((p1, o1), (p2, o2), (p3, o3)):
        r = jnp.dot(oh, p[...], preferred_element_type=jnp.float32)
        o[...] = r.reshape(_FIX_GRP, _SEQ_TCP, _D)


def _fix_tail(zfix, o1, o2, o3, p1, p2, p3):
    ngrp = _BATCH // _FIX_GRP
    zp = jnp.pad(zfix, ((0, 0), (0, _SEQ_TCP - _SEQ_TC)))
    z3 = zp.reshape(ngrp, 1, _FIX_GRP * _SEQ_TCP)
    pp = [jnp.pad(p, ((0, _VPAD - _VOCAB), (0, 0))) for p in (p1, p2, p3)]
    any_spec = pl.BlockSpec(memory_space=pltpu.MemorySpace.HBM)
    out_spec = pl.BlockSpec((_FIX_GRP, _SEQ_TCP, _D),
                            lambda i: (i, _SEQ_SC // _SEQ_TCP, 0))
    res = pl.pallas_call(
        _fix_body,
        grid=(ngrp,),
        in_specs=[
            pl.BlockSpec((1, 1, _FIX_GRP * _SEQ_TCP), lambda i: (i, 0, 0)),
            any_spec, any_spec, any_spec,
            pl.BlockSpec((_VPAD, _D), lambda i: (0, 0)),
            pl.BlockSpec((_VPAD, _D), lambda i: (0, 0)),
            pl.BlockSpec((_VPAD, _D), lambda i: (0, 0)),
        ],
        out_specs=[out_spec] * 3,
        out_shape=[jax.ShapeDtypeStruct((_BATCH, _SEQ, _D), jnp.float32)] * 3,
        input_output_aliases={1: 0, 2: 1, 3: 2},
    )(z3, o1, o2, o3, pp[0], pp[1], pp[2])
    return tuple(res)


def kernel(Z, table_mat2vec, table_magpie, table_oliynyk,
           W_mat2vec, b_mat2vec, W_magpie, b_magpie, W_oliynyk, b_oliynyk):
    p1, p2, p3 = _project_tables(
        table_mat2vec, W_mat2vec, b_mat2vec,
        table_magpie, W_magpie, b_magpie,
        table_oliynyk, W_oliynyk, b_oliynyk)
    o1, o2, o3 = _gather_sc(p1, p2, p3, Z[:, :_SEQ_SC])
    return _fix_tail(Z[:, _SEQ_SC:], o1, o2, o3, p1, p2, p3)


# speedup vs baseline: 1.0677x; 1.0434x over previous
import functools

import jax
import jax.numpy as jnp
from jax import lax
from jax.experimental import pallas as pl
from jax.experimental.pallas import tpu as pltpu
from jax.experimental.pallas import tpu_sc as plsc

_VOCAB = 119
_VPAD = 128
_D = 512
_BATCH, _SEQ = 4096, 50
_SEQ_SC = 48
_SEQ_TC = _SEQ - _SEQ_SC
_SEQ_TCP = 8

_NC, _NS = 2, 16
_NW = _NC * _NS
_B_PER_W = _BATCH // _NW

_FIX_GRP = 64


def _proj_body(t1, w1, b1, t2, w2, b2, t3, w3, b3, o1, o2, o3):
    o1[...] = jnp.dot(t1[...], w1[...], preferred_element_type=jnp.float32) + b1[...]
    o2[...] = jnp.dot(t2[...], w2[...], preferred_element_type=jnp.float32) + b2[...]
    o3[...] = jnp.dot(t3[...], w3[...], preferred_element_type=jnp.float32) + b3[...]


def _project_tables(t1, w1, b1, t2, w2, b2, t3, w3, b3):
    out = [jax.ShapeDtypeStruct((_VOCAB, _D), jnp.float32)] * 3
    return pl.pallas_call(_proj_body, out_shape=out)(
        t1, w1, b1.reshape(1, _D), t2, w2, b2.reshape(1, _D),
        t3, w3, b3.reshape(1, _D))


_mesh = plsc.VectorSubcoreMesh(core_axis_name="c", subcore_axis_name="s")


@functools.partial(
    pl.kernel,
    mesh=_mesh,
    out_type=[jax.ShapeDtypeStruct((_BATCH, _SEQ, _D), jnp.float32)] * 3,
    scratch_types=[
        pltpu.VMEM((_B_PER_W, _SEQ_SC), jnp.int32),
        pltpu.VMEM((_SEQ_SC, _D), jnp.float32),
        pltpu.VMEM((_SEQ_SC, _D), jnp.float32),
        pltpu.VMEM((_SEQ_SC, _D), jnp.float32),
        pltpu.SemaphoreType.DMA,
        pltpu.SemaphoreType.DMA,
    ],
    compiler_params=pltpu.CompilerParams(use_tc_tiling_on_sc=True),
)
def _gather_sc(p1, p2, p3, idx_hbm, o1, o2, o3,
               idx_v, r0, r1, r2, gsem, wsem):
    wid = lax.axis_index("s") * _NC + lax.axis_index("c")
    tbls = (p1, p2, p3)
    outs = (o1, o2, o3)
    bufs = (r0, r1, r2)

    b0 = wid * _B_PER_W
    pltpu.sync_copy(idx_hbm.at[pl.ds(b0, _B_PER_W)], idx_v)

    def batch_body(b, carry):
        idx_c = idx_v.at[b]
        for t in range(3):
            @pl.when(b > 0)
            def _drain():
                pltpu.make_async_copy(
                    bufs[t], outs[t].at[0, pl.ds(0, _SEQ_SC)], wsem).wait()

            pltpu.async_copy(tbls[t].at[idx_c], bufs[t], gsem).wait()
            pltpu.async_copy(
                bufs[t], outs[t].at[b0 + b, pl.ds(0, _SEQ_SC)], wsem)
        return carry

    lax.fori_loop(0, _B_PER_W, batch_body, 0)
    for t in range(3):
        pltpu.make_async_copy(
            bufs[t], outs[t].at[0, pl.ds(0, _SEQ_SC)], wsem).wait()


def _fix_body(zb, p1, p2, p3, o1, o2, o3):
    zcol = zb[0].reshape(_FIX_GRP * _SEQ_TC, 1)
    iota_v = lax.broadcasted_iota(jnp.int32, (_FIX_GRP * _SEQ_TC, _VPAD), 1)
    oh = (zcol == iota_v).astype(jnp.float32)
    for p, o in ((p1, o1), (p2, o2), (p3, o3)):
        r = jnp.dot(oh, p[...], preferred_element_type=jnp.float32)
        o[...] = r.reshape(_FIX_GRP, _SEQ_TC, _D)


def _fix_tail(zfix, o1, o2, o3, p1, p2, p3):
    ngrp = _BATCH // _FIX_GRP
    z3 = zfix.reshape(ngrp, 1, _FIX_GRP * _SEQ_TC)
    pp = [jnp.pad(p, ((0, _VPAD - _VOCAB), (0, 0))) for p in (p1, p2, p3)]
    out_spec = pl.BlockSpec((_FIX_GRP, _SEQ_TC, _D), lambda i: (i, 0, 0))
    tails = pl.pallas_call(
        _fix_body,
        grid=(ngrp,),
        in_specs=[
            pl.BlockSpec((1, 1, _FIX_GRP * _SEQ_TC), lambda i: (i, 0, 0)),
            pl.BlockSpec((_VPAD, _D), lambda i: (0, 0)),
            pl.BlockSpec((_VPAD, _D), lambda i: (0, 0)),
            pl.BlockSpec((_VPAD, _D), lambda i: (0, 0)),
        ],
        out_specs=[out_spec] * 3,
        out_shape=[jax.ShapeDtypeStruct((_BATCH, _SEQ_TC, _D), jnp.float32)] * 3,
    )(z3, pp[0], pp[1], pp[2])
    return tuple(
        lax.dynamic_update_slice(o, t, (0, _SEQ_SC, 0))
        for o, t in zip((o1, o2, o3), tails))


def kernel(Z, table_mat2vec, table_magpie, table_oliynyk,
           W_mat2vec, b_mat2vec, W_magpie, b_magpie, W_oliynyk, b_oliynyk):
    p1, p2, p3 = _project_tables(
        table_mat2vec, W_mat2vec, b_mat2vec,
        table_magpie, W_magpie, b_magpie,
        table_oliynyk, W_oliynyk, b_oliynyk)
    o1, o2, o3 = _gather_sc(p1, p2, p3, Z[:, :_SEQ_SC])
    return _fix_tail(Z[:, _SEQ_SC:], o1, o2, o3, p1, p2, p3)
